# baseline (speedup 1.0000x reference)
import functools

import jax
import jax.numpy as jnp
from jax import lax
from jax.experimental import pallas as pl
from jax.experimental.pallas import tpu as pltpu
from jax.experimental.pallas import tpu_sc as plsc

_NC = 2
_NS = 16
_NW = _NC * _NS
_NBUF = 8
_LANE = 128


@functools.lru_cache(maxsize=None)
def _make_gather(B, D):
    b_per_w = B // _NW
    n_chunks = b_per_w // _NBUF
    mesh = plsc.VectorSubcoreMesh(core_axis_name="c", subcore_axis_name="s")

    @functools.partial(
        pl.kernel,
        mesh=mesh,
        compiler_params=pltpu.CompilerParams(needs_layout_passes=False),
        out_type=(
            jax.ShapeDtypeStruct((D, B), jnp.float32),
            jax.ShapeDtypeStruct((D, B), jnp.float32),
        ),
        scratch_types=[
            pltpu.VMEM((b_per_w + 16,), jnp.int32),
            pltpu.VMEM((b_per_w + 16,), jnp.int32),
            pltpu.VMEM((_NBUF, D, _LANE), jnp.float32),
            pltpu.VMEM((_NBUF, D, _LANE), jnp.float32),
            pltpu.VMEM((D, b_per_w), jnp.float32),
            pltpu.VMEM((D, b_per_w), jnp.float32),
            [pltpu.SemaphoreType.DMA] * _NBUF,
        ],
    )
    def gather_kernel(uid_hbm, iid_hbm, wuT_hbm, wiT_hbm, ueT_hbm, ieT_hbm,
                      uidx_v, iidx_v, ulb_v, ilb_v,
                      urows_v, irows_v, sems):
        wid = lax.axis_index("s") * _NC + lax.axis_index("c")
        base = wid * b_per_w
        pltpu.sync_copy(uid_hbm.at[pl.ds(base, b_per_w)],
                        uidx_v.at[pl.ds(0, b_per_w)])
        pltpu.sync_copy(iid_hbm.at[pl.ds(base, b_per_w)],
                        iidx_v.at[pl.ds(0, b_per_w)])

        rows = lax.iota(jnp.int32, 16)

        def fire(u, v, b):
            uo = pl.multiple_of((u // _LANE) * _LANE, _LANE)
            pltpu.async_copy(wuT_hbm.at[:, pl.ds(uo, _LANE)],
                             ulb_v.at[b], sems[b])
            vo = pl.multiple_of((v // _LANE) * _LANE, _LANE)
            pltpu.async_copy(wiT_hbm.at[:, pl.ds(vo, _LANE)],
                             ilb_v.at[b], sems[b])

        def extract(lb, col, out, i):
            coli = jnp.full((16,), col % _LANE, jnp.int32)
            outi = jnp.full((16,), i, jnp.int32)
            for half in range(D // 16):
                r = rows + (16 * half)
                x = plsc.load_gather(lb, [r, coli])
                plsc.store_scatter(out, [r, outi], x)

        def drain_slot(b):
            pltpu.make_async_copy(wuT_hbm.at[:, pl.ds(0, _LANE)],
                                  ulb_v.at[b], sems[b]).wait()
            pltpu.make_async_copy(wiT_hbm.at[:, pl.ds(0, _LANE)],
                                  ilb_v.at[b], sems[b]).wait()

        uv0_p = uidx_v[pl.ds(0, 16)]
        iv0_p = iidx_v[pl.ds(0, 16)]
        for b in range(_NBUF):
            fire(uv0_p[b], iv0_p[b], b)

        def chunk_body(c, carry):
            o = pl.multiple_of(c * 16, 16)
            uv0 = uidx_v[pl.ds(o, 16)]
            iv0 = iidx_v[pl.ds(o, 16)]
            o1 = pl.multiple_of(c * 16 + 16, 16)
            uv1 = uidx_v[pl.ds(o1, 16)]
            iv1 = iidx_v[pl.ds(o1, 16)]
            for b in range(_NBUF):
                i = c * 16 + b
                drain_slot(b)
                extract(ulb_v.at[b], uv0[b], urows_v, i)
                extract(ilb_v.at[b], iv0[b], irows_v, i)
                fire(uv0[b + 8], iv0[b + 8], b)
            for b in range(_NBUF):
                i = c * 16 + 8 + b
                drain_slot(b)
                extract(ulb_v.at[b], uv0[b + 8], urows_v, i)
                extract(ilb_v.at[b], iv0[b + 8], irows_v, i)

                @pl.when(i + 8 < b_per_w)
                def _():
                    fire(uv1[b], iv1[b], b)
            return carry

        lax.fori_loop(0, b_per_w // 16, chunk_body, 0)

        pltpu.sync_copy(urows_v, ueT_hbm.at[:, pl.ds(base, b_per_w)])
        pltpu.sync_copy(irows_v, ieT_hbm.at[:, pl.ds(base, b_per_w)])

    return gather_kernel


@functools.lru_cache(maxsize=None)
def _make_mlp(B, D, H1, H2, bb):
    grid = (B // bb,)

    def mlp_kernel(ueT_ref, ieT_ref, w1T_ref, b1_ref, w2T_ref, b2_ref,
                   wo_ref, bo_ref, out_ref):
        hT = (
            jnp.dot(w1T_ref[:, :D], ueT_ref[...], preferred_element_type=jnp.float32)
            + jnp.dot(w1T_ref[:, D:], ieT_ref[...], preferred_element_type=jnp.float32)
            + b1_ref[...]
        )
        hT = jnp.maximum(hT, 0.0)
        h2T = jnp.dot(w2T_ref[...], hT, preferred_element_type=jnp.float32) + b2_ref[...]
        h2T = jnp.maximum(h2T, 0.0)
        logits = jnp.sum(h2T * wo_ref[...], axis=0) + bo_ref[0, 0]
        out_ref[...] = jax.nn.sigmoid(logits)

    return pl.pallas_call(
        mlp_kernel,
        grid=grid,
        in_specs=[
            pl.BlockSpec((D, bb), lambda i: (0, i)),
            pl.BlockSpec((D, bb), lambda i: (0, i)),
            pl.BlockSpec((H1, 2 * D), lambda i: (0, 0)),
            pl.BlockSpec((H1, 1), lambda i: (0, 0)),
            pl.BlockSpec((H2, H1), lambda i: (0, 0)),
            pl.BlockSpec((H2, 1), lambda i: (0, 0)),
            pl.BlockSpec((H2, 1), lambda i: (0, 0)),
            pl.BlockSpec((1, 1), lambda i: (0, 0), memory_space=pltpu.SMEM),
        ],
        out_specs=pl.BlockSpec((bb,), lambda i: (i,)),
        out_shape=jax.ShapeDtypeStruct((B,), jnp.float32),
    )


def kernel(user_ids, item_ids, Wu, Wi, W1, b1, W2, b2, Wo, bo):
    B = user_ids.shape[0]
    D = Wu.shape[1]
    H1 = W1.shape[1]
    H2 = W2.shape[1]
    uid = user_ids.astype(jnp.int32)
    iid = item_ids.astype(jnp.int32)
    n_parts = 2
    bp = B // n_parts
    bb = 2048
    gather = _make_gather(bp, D)
    mlp = _make_mlp(bp, D, H1, H2, bb)
    WuT, WiT, W1T, W2T = Wu.T, Wi.T, W1.T, W2.T
    b1c = b1.reshape(H1, 1)
    b2c = b2.reshape(H2, 1)
    boc = bo.reshape(1, 1)
    outs = []
    for p in range(n_parts):
        ueT, ieT = gather(uid[p * bp:(p + 1) * bp], iid[p * bp:(p + 1) * bp],
                          WuT, WiT)
        outs.append(mlp(ueT, ieT, W1T, b1c, W2T, b2c, Wo, boc))
    return jnp.concatenate(outs)


# speedup vs baseline: 1.0411x; 1.0411x over previous
import functools

import jax
import jax.numpy as jnp
from jax import lax
from jax.experimental import pallas as pl
from jax.experimental.pallas import tpu as pltpu
from jax.experimental.pallas import tpu_sc as plsc

_NC = 2
_NS = 16
_NW = _NC * _NS
_NBUF = 8
_LANE = 128


@functools.lru_cache(maxsize=None)
def _make_gather(B, D):
    b_per_w = B // _NW
    n_chunks = b_per_w // _NBUF
    mesh = plsc.VectorSubcoreMesh(core_axis_name="c", subcore_axis_name="s")

    @functools.partial(
        pl.kernel,
        mesh=mesh,
        compiler_params=pltpu.CompilerParams(needs_layout_passes=False),
        out_type=(
            jax.ShapeDtypeStruct((D, B), jnp.float32),
            jax.ShapeDtypeStruct((D, B), jnp.float32),
        ),
        scratch_types=[
            pltpu.VMEM((b_per_w + 16,), jnp.int32),
            pltpu.VMEM((b_per_w + 16,), jnp.int32),
            pltpu.VMEM((_NBUF, D, _LANE), jnp.float32),
            pltpu.VMEM((_NBUF, D, _LANE), jnp.float32),
            pltpu.VMEM((D, b_per_w), jnp.float32),
            pltpu.VMEM((D, b_per_w), jnp.float32),
            [pltpu.SemaphoreType.DMA] * _NBUF,
        ],
    )
    def gather_kernel(uid_hbm, iid_hbm, wuT_hbm, wiT_hbm, ueT_hbm, ieT_hbm,
                      uidx_v, iidx_v, ulb_v, ilb_v,
                      urows_v, irows_v, sems):
        wid = lax.axis_index("s") * _NC + lax.axis_index("c")
        base = wid * b_per_w
        pltpu.sync_copy(uid_hbm.at[pl.ds(base, b_per_w)],
                        uidx_v.at[pl.ds(0, b_per_w)])
        pltpu.sync_copy(iid_hbm.at[pl.ds(base, b_per_w)],
                        iidx_v.at[pl.ds(0, b_per_w)])

        rows = lax.iota(jnp.int32, 16)

        def fire(u, v, b):
            uo = pl.multiple_of((u // _LANE) * _LANE, _LANE)
            pltpu.async_copy(wuT_hbm.at[:, pl.ds(uo, _LANE)],
                             ulb_v.at[b], sems[b])
            vo = pl.multiple_of((v // _LANE) * _LANE, _LANE)
            pltpu.async_copy(wiT_hbm.at[:, pl.ds(vo, _LANE)],
                             ilb_v.at[b], sems[b])

        def extract(lb, col, out, i):
            coli = jnp.full((16,), col % _LANE, jnp.int32)
            outi = jnp.full((16,), i, jnp.int32)
            for half in range(D // 16):
                r = rows + (16 * half)
                x = plsc.load_gather(lb, [r, coli])
                plsc.store_scatter(out, [r, outi], x)

        def drain_slot(b):
            pltpu.make_async_copy(wuT_hbm.at[:, pl.ds(0, _LANE)],
                                  ulb_v.at[b], sems[b]).wait()
            pltpu.make_async_copy(wiT_hbm.at[:, pl.ds(0, _LANE)],
                                  ilb_v.at[b], sems[b]).wait()

        uv0_p = uidx_v[pl.ds(0, 16)]
        iv0_p = iidx_v[pl.ds(0, 16)]
        for b in range(_NBUF):
            fire(uv0_p[b], iv0_p[b], b)

        def chunk_body(c, carry):
            o = pl.multiple_of(c * 16, 16)
            uv0 = uidx_v[pl.ds(o, 16)]
            iv0 = iidx_v[pl.ds(o, 16)]
            o1 = pl.multiple_of(c * 16 + 16, 16)
            uv1 = uidx_v[pl.ds(o1, 16)]
            iv1 = iidx_v[pl.ds(o1, 16)]
            for b in range(_NBUF):
                i = c * 16 + b
                drain_slot(b)
                extract(ulb_v.at[b], uv0[b], urows_v, i)
                extract(ilb_v.at[b], iv0[b], irows_v, i)
                fire(uv0[b + 8], iv0[b + 8], b)
            for b in range(_NBUF):
                i = c * 16 + 8 + b
                drain_slot(b)
                extract(ulb_v.at[b], uv0[b + 8], urows_v, i)
                extract(ilb_v.at[b], iv0[b + 8], irows_v, i)

                @pl.when(i + 8 < b_per_w)
                def _():
                    fire(uv1[b], iv1[b], b)
            return carry

        lax.fori_loop(0, b_per_w // 16, chunk_body, 0)

        pltpu.sync_copy(urows_v, ueT_hbm.at[:, pl.ds(base, b_per_w)])
        pltpu.sync_copy(irows_v, ieT_hbm.at[:, pl.ds(base, b_per_w)])

    return gather_kernel


@functools.lru_cache(maxsize=None)
def _make_mlp(B, D, H1, H2, bb):
    grid = (B // bb,)

    def mlp_kernel(ueT_ref, ieT_ref, w1T_ref, b1_ref, w2T_ref, b2_ref,
                   wo_ref, bo_ref, out_ref):
        hT = (
            jnp.dot(w1T_ref[:, :D], ueT_ref[...], preferred_element_type=jnp.float32)
            + jnp.dot(w1T_ref[:, D:], ieT_ref[...], preferred_element_type=jnp.float32)
            + b1_ref[...]
        )
        hT = jnp.maximum(hT, 0.0)
        h2T = jnp.dot(w2T_ref[...], hT, preferred_element_type=jnp.float32) + b2_ref[...]
        h2T = jnp.maximum(h2T, 0.0)
        logits = jnp.sum(h2T * wo_ref[...], axis=0) + bo_ref[0, 0]
        out_ref[...] = jax.nn.sigmoid(logits)

    return pl.pallas_call(
        mlp_kernel,
        grid=grid,
        in_specs=[
            pl.BlockSpec((D, bb), lambda i: (0, i)),
            pl.BlockSpec((D, bb), lambda i: (0, i)),
            pl.BlockSpec((H1, 2 * D), lambda i: (0, 0)),
            pl.BlockSpec((H1, 1), lambda i: (0, 0)),
            pl.BlockSpec((H2, H1), lambda i: (0, 0)),
            pl.BlockSpec((H2, 1), lambda i: (0, 0)),
            pl.BlockSpec((H2, 1), lambda i: (0, 0)),
            pl.BlockSpec((1, 1), lambda i: (0, 0), memory_space=pltpu.SMEM),
        ],
        out_specs=pl.BlockSpec((bb,), lambda i: (i,)),
        out_shape=jax.ShapeDtypeStruct((B,), jnp.float32),
    )


def kernel(user_ids, item_ids, Wu, Wi, W1, b1, W2, b2, Wo, bo):
    B = user_ids.shape[0]
    D = Wu.shape[1]
    H1 = W1.shape[1]
    H2 = W2.shape[1]
    uid = user_ids.astype(jnp.int32)
    iid = item_ids.astype(jnp.int32)
    n_parts = 1
    bp = B // n_parts
    bb = 4096
    gather = _make_gather(bp, D)
    mlp = _make_mlp(bp, D, H1, H2, bb)
    WuT, WiT, W1T, W2T = Wu.T, Wi.T, W1.T, W2.T
    b1c = b1.reshape(H1, 1)
    b2c = b2.reshape(H2, 1)
    boc = bo.reshape(1, 1)
    outs = []
    for p in range(n_parts):
        ueT, ieT = gather(uid[p * bp:(p + 1) * bp], iid[p * bp:(p + 1) * bp],
                          WuT, WiT)
        outs.append(mlp(ueT, ieT, W1T, b1c, W2T, b2c, Wo, boc))
    return jnp.concatenate(outs)
